# baseline (speedup 1.0000x reference)
import functools
import jax
import jax.numpy as jnp
from jax import lax
from jax.experimental import pallas as pl
from jax.experimental.pallas import tpu as pltpu
from jax.experimental.pallas import tpu_sc as plsc

_NC = 2
_NS = 16
_NW = _NC * _NS
_RU = 5
_B_SC = 16
_BB = 4


def _sc_half(daytime, W_day, W_time, W_node, E):
    B, L, _ = daytime.shape
    N, NODE = W_node.shape
    DS = W_day.shape[1]
    TS = W_time.shape[1]
    P = B * L
    PPW = P // _NW

    day_idx = daytime[..., 0].reshape(_NW, PPW)
    time_idx = daytime[..., 1].reshape(_NW, PPW)
    W_day_p = jnp.pad(W_day, ((0, 0), (0, 128 - DS)))
    W_time_p = jnp.pad(W_time, ((0, 0), (0, 128 - TS)))
    W_node_p = jnp.pad(W_node, ((0, 0), (0, E - NODE)))

    mesh = plsc.VectorSubcoreMesh(core_axis_name="c", subcore_axis_name="s")

    @functools.partial(
        pl.kernel,
        mesh=mesh,
        out_type=jax.ShapeDtypeStruct((B, L, N, E), jnp.float32),
        scratch_types=[
            pltpu.VMEM((PPW,), jnp.int32),
            pltpu.VMEM((PPW,), jnp.int32),
            pltpu.VMEM((PPW, 128), jnp.float32),
            pltpu.VMEM((PPW, 128), jnp.float32),
            pltpu.VMEM((N, E), jnp.float32),
            pltpu.VMEM((N, E), jnp.float32),
            pltpu.SemaphoreType.DMA,
            pltpu.SemaphoreType.DMA,
            pltpu.SemaphoreType.DMA,
        ],
    )
    def sc_k(dayi_hbm, timei_hbm, wday_hbm, wtime_hbm, wnode_hbm, out_hbm,
             dayi_v, timei_v, dayrows_v, timerows_v, tile0, tile1,
             sem0, sem1, gsem):
        wid = lax.axis_index("s") * _NC + lax.axis_index("c")
        base = wid * PPW
        pltpu.sync_copy(dayi_hbm.at[wid], dayi_v)
        pltpu.sync_copy(timei_hbm.at[wid], timei_v)
        pltpu.async_copy(wday_hbm.at[dayi_v], dayrows_v, gsem).wait()
        pltpu.async_copy(wtime_hbm.at[timei_v], timerows_v, gsem).wait()
        pltpu.sync_copy(wnode_hbm, tile0)
        pltpu.sync_copy(wnode_hbm, tile1)

        def build(tile_v, j):
            d0 = dayrows_v[j, 0:16]
            d1 = dayrows_v[j, 16:32]
            t0 = timerows_v[j, 0:16]
            t1 = timerows_v[j, 16:32]

            def row_body(i, c):
                r = i * _RU
                for k in range(_RU):
                    tile_v[r + k, NODE:NODE + 16] = d0
                    tile_v[r + k, NODE + 16:NODE + 32] = d1
                    tile_v[r + k, NODE + 32:NODE + 48] = t0
                    tile_v[r + k, NODE + 48:NODE + 64] = t1
                return c

            lax.fori_loop(0, N // _RU, row_body, 0)

        def fire(tile_v, sem, j):
            k = base + j
            bi = k // L
            li = k - bi * L
            pltpu.async_copy(tile_v, out_hbm.at[bi, li], sem)

        def drain(tile_v, sem):
            pltpu.make_async_copy(tile_v, out_hbm.at[0, 0], sem).wait()

        def body(i, carry):
            j0 = 2 * i

            @pl.when(i > 0)
            def _():
                drain(tile0, sem0)

            build(tile0, j0)
            fire(tile0, sem0, j0)

            @pl.when(i > 0)
            def _():
                drain(tile1, sem1)

            build(tile1, j0 + 1)
            fire(tile1, sem1, j0 + 1)
            return carry

        lax.fori_loop(0, PPW // 2, body, 0)
        drain(tile0, sem0)
        drain(tile1, sem1)

    return sc_k(day_idx, time_idx, W_day_p, W_time_p, W_node_p)


def _tc_body(idx_ref, wday_ref, wtime_ref, wnode_ref, out_ref):
    g = pl.program_id(0)
    L = out_ref.shape[1]
    N, NS = wnode_ref.shape
    DS = wday_ref.shape[1]
    TS = wtime_ref.shape[1]
    node = wnode_ref[...]
    for bb in range(_BB):
        b = g * _BB + bb
        for l in range(L):
            d = idx_ref[b, l, 0]
            t = idx_ref[b, l, 1]
            day_b = jnp.broadcast_to(wday_ref[d, :][None, :], (N, DS))
            time_b = jnp.broadcast_to(wtime_ref[t, :][None, :], (N, TS))
            out_ref[bb, l] = jnp.concatenate([node, day_b, time_b], axis=-1)


def _tc_half(daytime, W_day, W_time, W_node, E):
    B, L, _ = daytime.shape
    N, NS = W_node.shape

    grid_spec = pltpu.PrefetchScalarGridSpec(
        num_scalar_prefetch=1,
        grid=(B // _BB,),
        in_specs=[
            pl.BlockSpec(W_day.shape, lambda b, idx: (0, 0)),
            pl.BlockSpec(W_time.shape, lambda b, idx: (0, 0)),
            pl.BlockSpec(W_node.shape, lambda b, idx: (0, 0)),
        ],
        out_specs=pl.BlockSpec((_BB, L, N, E), lambda b, idx: (b, 0, 0, 0)),
    )
    return pl.pallas_call(
        _tc_body,
        grid_spec=grid_spec,
        out_shape=jax.ShapeDtypeStruct((B, L, N, E), jnp.float32),
    )(daytime, W_day, W_time, W_node)


def kernel(daytime, W_day, W_time, W_node):
    B, L, _ = daytime.shape
    N, NS = W_node.shape
    E = NS + W_day.shape[1] + W_time.shape[1]
    sc_out = _sc_half(daytime[:_B_SC], W_day, W_time, W_node, E)
    tc_out = _tc_half(daytime[_B_SC:], W_day, W_time, W_node, E)
    return jnp.concatenate([sc_out, tc_out], axis=0)


# speedup vs baseline: 1.6980x; 1.6980x over previous
import functools
import jax
import jax.numpy as jnp
from jax import lax
from jax.experimental import pallas as pl
from jax.experimental.pallas import tpu as pltpu
from jax.experimental.pallas import tpu_sc as plsc

_NC = 2
_NS = 16
_NW = _NC * _NS
_RU = 5


def kernel(daytime, W_day, W_time, W_node):
    B, L, _ = daytime.shape
    N, NODE = W_node.shape
    DS = W_day.shape[1]
    TS = W_time.shape[1]
    E = NODE + DS + TS
    P = B * L
    PPW = P // _NW

    day_idx = daytime[..., 0].reshape(_NW, PPW)
    time_idx = daytime[..., 1].reshape(_NW, PPW)
    W_day_p = jnp.pad(W_day, ((0, 0), (0, 128 - DS)))
    W_time_p = jnp.pad(W_time, ((0, 0), (0, 128 - TS)))
    W_node_p = jnp.pad(W_node, ((0, 0), (0, E - NODE)))

    mesh = plsc.VectorSubcoreMesh(core_axis_name="c", subcore_axis_name="s")

    @functools.partial(
        pl.kernel,
        mesh=mesh,
        out_type=jax.ShapeDtypeStruct((B, L, N, E), jnp.float32),
        scratch_types=[
            pltpu.VMEM((PPW,), jnp.int32),
            pltpu.VMEM((PPW,), jnp.int32),
            pltpu.VMEM((PPW, 128), jnp.float32),
            pltpu.VMEM((PPW, 128), jnp.float32),
            pltpu.VMEM((N, E), jnp.float32),
            pltpu.VMEM((N, E), jnp.float32),
            pltpu.SemaphoreType.DMA,
            pltpu.SemaphoreType.DMA,
            pltpu.SemaphoreType.DMA,
        ],
    )
    def sc_k(dayi_hbm, timei_hbm, wday_hbm, wtime_hbm, wnode_hbm, out_hbm,
             dayi_v, timei_v, dayrows_v, timerows_v, tile0, tile1,
             sem0, sem1, gsem):
        wid = lax.axis_index("s") * _NC + lax.axis_index("c")
        base = wid * PPW
        pltpu.sync_copy(dayi_hbm.at[wid], dayi_v)
        pltpu.sync_copy(timei_hbm.at[wid], timei_v)
        pltpu.async_copy(wday_hbm.at[dayi_v], dayrows_v, gsem).wait()
        pltpu.async_copy(wtime_hbm.at[timei_v], timerows_v, gsem).wait()
        pltpu.sync_copy(wnode_hbm, tile0)
        pltpu.sync_copy(wnode_hbm, tile1)

        def build(tile_v, j):
            d0 = dayrows_v[j, 0:16]
            d1 = dayrows_v[j, 16:32]
            t0 = timerows_v[j, 0:16]
            t1 = timerows_v[j, 16:32]

            def row_body(i, c):
                r = i * _RU
                for k in range(_RU):
                    tile_v[r + k, NODE:NODE + 16] = d0
                    tile_v[r + k, NODE + 16:NODE + 32] = d1
                    tile_v[r + k, NODE + 32:NODE + 48] = t0
                    tile_v[r + k, NODE + 48:NODE + 64] = t1
                return c

            lax.fori_loop(0, N // _RU, row_body, 0)

        def fire(tile_v, sem, j):
            k = base + j
            bi = k // L
            li = k - bi * L
            pltpu.async_copy(tile_v, out_hbm.at[bi, li], sem)

        def drain(tile_v, sem):
            pltpu.make_async_copy(tile_v, out_hbm.at[0, 0], sem).wait()

        def body(i, carry):
            j0 = 2 * i

            @pl.when(i > 0)
            def _():
                drain(tile0, sem0)

            build(tile0, j0)
            fire(tile0, sem0, j0)

            @pl.when(i > 0)
            def _():
                drain(tile1, sem1)

            build(tile1, j0 + 1)
            fire(tile1, sem1, j0 + 1)
            return carry

        lax.fori_loop(0, PPW // 2, body, 0)
        drain(tile0, sem0)
        drain(tile1, sem1)

    return sc_k(day_idx, time_idx, W_day_p, W_time_p, W_node_p)


# speedup vs baseline: 1.7811x; 1.0490x over previous
import functools
import jax
import jax.numpy as jnp
from jax import lax
from jax.experimental import pallas as pl
from jax.experimental.pallas import tpu as pltpu
from jax.experimental.pallas import tpu_sc as plsc

_NC = 2
_NS = 16
_NW = _NC * _NS
_RU = 5


def kernel(daytime, W_day, W_time, W_node):
    B, L, _ = daytime.shape
    N, NODE = W_node.shape
    DS = W_day.shape[1]
    TS = W_time.shape[1]
    E = NODE + DS + TS
    P = B * L
    PPW = P // _NW

    day_idx = daytime[..., 0].reshape(_NW, PPW)
    time_idx = daytime[..., 1].reshape(_NW, PPW)
    W_day_p = jnp.pad(W_day, ((0, 0), (0, 128 - DS)))
    W_time_p = jnp.pad(W_time, ((0, 0), (0, 128 - TS)))
    W_node_p = jnp.pad(W_node, ((0, 0), (0, E - NODE)))

    mesh = plsc.VectorSubcoreMesh(core_axis_name="c", subcore_axis_name="s")

    @functools.partial(
        pl.kernel,
        mesh=mesh,
        out_type=jax.ShapeDtypeStruct((B, L, N, E), jnp.float32),
        scratch_types=[
            pltpu.VMEM((PPW,), jnp.int32),
            pltpu.VMEM((PPW,), jnp.int32),
            pltpu.VMEM((PPW, 128), jnp.float32),
            pltpu.VMEM((PPW, 128), jnp.float32),
            pltpu.VMEM((N, E), jnp.float32),
            pltpu.VMEM((N, E), jnp.float32),
            pltpu.SemaphoreType.DMA,
            pltpu.SemaphoreType.DMA,
            pltpu.SemaphoreType.DMA,
        ],
    )
    def sc_k(dayi_hbm, timei_hbm, wday_hbm, wtime_hbm, wnode_hbm, out_hbm,
             dayi_v, timei_v, dayrows_v, timerows_v, tile0, tile1,
             sem0, sem1, gsem):
        wid = lax.axis_index("s") * _NC + lax.axis_index("c")
        base = wid * PPW
        pltpu.sync_copy(dayi_hbm.at[wid], dayi_v)
        pltpu.sync_copy(timei_hbm.at[wid], timei_v)
        n0 = pltpu.async_copy(wnode_hbm, tile0, sem0)
        n1 = pltpu.async_copy(wnode_hbm, tile1, sem1)
        gd = pltpu.async_copy(wday_hbm.at[dayi_v], dayrows_v, gsem)
        gt = pltpu.async_copy(wtime_hbm.at[timei_v], timerows_v, gsem)
        gd.wait()
        gt.wait()
        n0.wait()
        n1.wait()

        def build(tile_v, j):
            d0 = dayrows_v[j, 0:16]
            d1 = dayrows_v[j, 16:32]
            t0 = timerows_v[j, 0:16]
            t1 = timerows_v[j, 16:32]

            def row_body(i, c):
                r = i * _RU
                for k in range(_RU):
                    tile_v[r + k, NODE:NODE + 16] = d0
                    tile_v[r + k, NODE + 16:NODE + 32] = d1
                    tile_v[r + k, NODE + 32:NODE + 48] = t0
                    tile_v[r + k, NODE + 48:NODE + 64] = t1
                return c

            lax.fori_loop(0, N // _RU, row_body, 0)

        def fire(tile_v, sem, j):
            k = base + j
            bi = k // L
            li = k - bi * L
            pltpu.async_copy(tile_v, out_hbm.at[bi, li], sem)

        def drain(tile_v, sem):
            pltpu.make_async_copy(tile_v, out_hbm.at[0, 0], sem).wait()

        def body(i, carry):
            j0 = 2 * i

            @pl.when(i > 0)
            def _():
                drain(tile0, sem0)

            build(tile0, j0)
            fire(tile0, sem0, j0)

            @pl.when(i > 0)
            def _():
                drain(tile1, sem1)

            build(tile1, j0 + 1)
            fire(tile1, sem1, j0 + 1)
            return carry

        lax.fori_loop(0, PPW // 2, body, 0)
        drain(tile0, sem0)
        drain(tile1, sem1)

    return sc_k(day_idx, time_idx, W_day_p, W_time_p, W_node_p)
